# baseline (speedup 1.0000x reference)
import functools

import jax
import jax.numpy as jnp
from jax.experimental import pallas as pl
import jax.experimental.pallas.tpu as pltpu

VOCAB = 256
EMB = 256
HID = 512
B = 32
S = 512
T = 128
G4 = 4 * HID


def _sigmoid(x):
    return 0.5 * jnp.tanh(0.5 * x) + 0.5


def _lstm_kernel(idx_ref, embed_ref, wih_t_ref, whh_t_ref, bias_ref,
                 whead_t_ref, bhead_ref,
                 out_ref, hn_ref, cn_ref,
                 ew_ref, gx_ref, lbuf_ref, h_ref, c_ref):
    k = pl.program_id(0)

    @pl.when(k == 0)
    def _init():
        ew_ref[...] = jnp.dot(embed_ref[...], wih_t_ref[...],
                              preferred_element_type=jnp.float32) + bias_ref[...]
        h_ref[...] = jnp.zeros((B, HID), jnp.float32)
        c_ref[...] = jnp.zeros((B, HID), jnp.float32)

    idx_tm = idx_ref[0]
    oh = (idx_tm[:, :, None] == jax.lax.broadcasted_iota(
        jnp.int32, (T, B, VOCAB), 2)).astype(jnp.float32)
    oh2 = oh.reshape(T * B, VOCAB)
    gx_ref[...] = jnp.dot(oh2, ew_ref[...], preferred_element_type=jnp.float32)

    whh_t = whh_t_ref[...]
    whead_t = whead_t_ref[...]
    bhead = bhead_ref[...]

    def step(t, carry):
        h, c = carry
        gates = gx_ref[pl.ds(t * B, B), :] + jnp.dot(
            h, whh_t, preferred_element_type=jnp.float32)
        i_g = _sigmoid(gates[:, 0 * HID:1 * HID])
        f_g = _sigmoid(gates[:, 1 * HID:2 * HID])
        g_g = jnp.tanh(gates[:, 2 * HID:3 * HID])
        o_g = _sigmoid(gates[:, 3 * HID:4 * HID])
        c_new = f_g * c + i_g * g_g
        h_new = o_g * jnp.tanh(c_new)
        mean = jnp.mean(h_new, axis=1, keepdims=True)
        cent = h_new - mean
        var = jnp.mean(cent * cent, axis=1, keepdims=True)
        normed = cent * jax.lax.rsqrt(var + 1e-5)
        logits_t = jnp.dot(normed, whead_t,
                           preferred_element_type=jnp.float32) + bhead
        lbuf_ref[pl.ds(t * B, B), :] = logits_t
        return h_new, c_new

    h_fin, c_fin = jax.lax.fori_loop(0, T, step, (h_ref[...], c_ref[...]),
                                     unroll=8)
    h_ref[...] = h_fin
    c_ref[...] = c_fin
    hn_ref[...] = h_fin
    cn_ref[...] = c_fin

    out_ref[...] = jnp.swapaxes(lbuf_ref[...].reshape(T, B, VOCAB), 0, 1)


@jax.jit
def kernel(idx, embed_table, W_ih, W_hh, b_ih, b_hh, ln_gamma, ln_beta,
           W_head, b_head):
    idx = idx.astype(jnp.int32)
    bias = (b_ih + b_hh).reshape(1, G4)
    grid = S // T

    out_tm, h_n, c_n = pl.pallas_call(
        _lstm_kernel,
        grid=(grid,),
        in_specs=[
            pl.BlockSpec((1, T, B), lambda k: (k, 0, 0)),
            pl.BlockSpec((VOCAB, EMB), lambda k: (0, 0)),
            pl.BlockSpec((EMB, G4), lambda k: (0, 0)),
            pl.BlockSpec((HID, G4), lambda k: (0, 0)),
            pl.BlockSpec((1, G4), lambda k: (0, 0)),
            pl.BlockSpec((HID, VOCAB), lambda k: (0, 0)),
            pl.BlockSpec((1, VOCAB), lambda k: (0, 0)),
        ],
        out_specs=[
            pl.BlockSpec((B, T, VOCAB), lambda k: (0, k, 0)),
            pl.BlockSpec((B, HID), lambda k: (0, 0)),
            pl.BlockSpec((B, HID), lambda k: (0, 0)),
        ],
        out_shape=[
            jax.ShapeDtypeStruct((B, S, VOCAB), jnp.float32),
            jax.ShapeDtypeStruct((B, HID), jnp.float32),
            jax.ShapeDtypeStruct((B, HID), jnp.float32),
        ],
        scratch_shapes=[
            pltpu.VMEM((VOCAB, G4), jnp.float32),
            pltpu.VMEM((T * B, G4), jnp.float32),
            pltpu.VMEM((T * B, VOCAB), jnp.float32),
            pltpu.VMEM((B, HID), jnp.float32),
            pltpu.VMEM((B, HID), jnp.float32),
        ],
        compiler_params=pltpu.CompilerParams(
            vmem_limit_bytes=100 * 1024 * 1024),
    )(jnp.swapaxes(idx, 0, 1).reshape(S // T, T, B), embed_table,
      W_ih.T, W_hh.T, bias,
      ln_gamma[:, None] * W_head.T,
      (b_head + ln_beta @ W_head.T).reshape(1, VOCAB))

    return (out_tm, h_n[None], c_n[None])


# speedup vs baseline: 1.3535x; 1.3535x over previous
import functools

import jax
import jax.numpy as jnp
from jax.experimental import pallas as pl
import jax.experimental.pallas.tpu as pltpu

VOCAB = 256
EMB = 256
HID = 512
B = 32
S = 512
T = 128
G4 = 4 * HID


def _sigmoid(x):
    return 0.5 * jnp.tanh(0.5 * x) + 0.5


def _lstm_kernel(idx_ref, embed_ref, wih_t_ref, whh_t_ref, bias_ref,
                 whead_t_ref, bhead_ref,
                 out_ref, hn_ref, cn_ref,
                 ew_ref, gx_ref, hbuf_ref, h_ref, c_ref):
    k = pl.program_id(0)

    @pl.when(k == 0)
    def _init():
        ew_ref[...] = jnp.dot(embed_ref[...], wih_t_ref[...],
                              preferred_element_type=jnp.float32) + bias_ref[...]
        h_ref[...] = jnp.zeros((B, HID), jnp.float32)
        c_ref[...] = jnp.zeros((B, HID), jnp.float32)

    idx_tm = idx_ref[0]
    oh = (idx_tm[:, :, None] == jax.lax.broadcasted_iota(
        jnp.int32, (T, B, VOCAB), 2)).astype(jnp.float32)
    oh2 = oh.reshape(T * B, VOCAB)
    gx_ref[...] = jnp.dot(oh2, ew_ref[...], preferred_element_type=jnp.float32)

    whh_t = whh_t_ref[...]

    def step(t, carry):
        h, c = carry
        gates = gx_ref[pl.ds(t * B, B), :] + jnp.dot(
            h, whh_t, preferred_element_type=jnp.float32)
        i_g = _sigmoid(gates[:, 0 * HID:1 * HID])
        f_g = _sigmoid(gates[:, 1 * HID:2 * HID])
        g_g = jnp.tanh(gates[:, 2 * HID:3 * HID])
        o_g = _sigmoid(gates[:, 3 * HID:4 * HID])
        c_new = f_g * c + i_g * g_g
        h_new = o_g * jnp.tanh(c_new)
        hbuf_ref[pl.ds(t * B, B), :] = h_new
        return h_new, c_new

    h_fin, c_fin = jax.lax.fori_loop(0, T, step, (h_ref[...], c_ref[...]),
                                     unroll=16)
    h_ref[...] = h_fin
    c_ref[...] = c_fin
    hn_ref[...] = h_fin
    cn_ref[...] = c_fin

    hb = hbuf_ref[...]
    mean = jnp.mean(hb, axis=1, keepdims=True)
    cent = hb - mean
    var = jnp.mean(cent * cent, axis=1, keepdims=True)
    normed = cent * jax.lax.rsqrt(var + 1e-5)
    logits = jnp.dot(normed, whead_t_ref[...],
                     preferred_element_type=jnp.float32) + bhead_ref[...]
    out_ref[...] = jnp.swapaxes(logits.reshape(T, B, VOCAB), 0, 1)


@jax.jit
def kernel(idx, embed_table, W_ih, W_hh, b_ih, b_hh, ln_gamma, ln_beta,
           W_head, b_head):
    idx = idx.astype(jnp.int32)
    bias = (b_ih + b_hh).reshape(1, G4)
    grid = S // T

    out_tm, h_n, c_n = pl.pallas_call(
        _lstm_kernel,
        grid=(grid,),
        in_specs=[
            pl.BlockSpec((1, T, B), lambda k: (k, 0, 0)),
            pl.BlockSpec((VOCAB, EMB), lambda k: (0, 0)),
            pl.BlockSpec((EMB, G4), lambda k: (0, 0)),
            pl.BlockSpec((HID, G4), lambda k: (0, 0)),
            pl.BlockSpec((1, G4), lambda k: (0, 0)),
            pl.BlockSpec((HID, VOCAB), lambda k: (0, 0)),
            pl.BlockSpec((1, VOCAB), lambda k: (0, 0)),
        ],
        out_specs=[
            pl.BlockSpec((B, T, VOCAB), lambda k: (0, k, 0)),
            pl.BlockSpec((B, HID), lambda k: (0, 0)),
            pl.BlockSpec((B, HID), lambda k: (0, 0)),
        ],
        out_shape=[
            jax.ShapeDtypeStruct((B, S, VOCAB), jnp.float32),
            jax.ShapeDtypeStruct((B, HID), jnp.float32),
            jax.ShapeDtypeStruct((B, HID), jnp.float32),
        ],
        scratch_shapes=[
            pltpu.VMEM((VOCAB, G4), jnp.float32),
            pltpu.VMEM((T * B, G4), jnp.float32),
            pltpu.VMEM((T * B, HID), jnp.float32),
            pltpu.VMEM((B, HID), jnp.float32),
            pltpu.VMEM((B, HID), jnp.float32),
        ],
        compiler_params=pltpu.CompilerParams(
            vmem_limit_bytes=100 * 1024 * 1024),
    )(jnp.swapaxes(idx, 0, 1).reshape(S // T, T, B), embed_table,
      W_ih.T, W_hh.T, bias,
      ln_gamma[:, None] * W_head.T,
      (b_head + ln_beta @ W_head.T).reshape(1, VOCAB))

    return (out_tm, h_n[None], c_n[None])
